# baseline (speedup 1.0000x reference)
import functools

import jax
import jax.numpy as jnp
from jax import lax
from jax.experimental import pallas as pl
from jax.experimental.pallas import tpu as pltpu
from jax.experimental.pallas import tpu_sc as plsc

_NUM_CORES = 2
_NUM_SUBCORES = 16
_NW = _NUM_CORES * _NUM_SUBCORES


def kernel(input, table):
    B, S = input.shape
    V, D = table.shape
    rows_per_w = B // _NW
    nbuf = 4
    n_groups = rows_per_w // nbuf

    mesh = plsc.VectorSubcoreMesh(core_axis_name="c", subcore_axis_name="s")

    @functools.partial(
        pl.kernel,
        mesh=mesh,
        out_type=jax.ShapeDtypeStruct((B, S, D), jnp.float32),
        scratch_types=(
            [pltpu.VMEM((rows_per_w, S), jnp.int32)]
            + [pltpu.VMEM((S, D), jnp.float32) for _ in range(nbuf)]
            + [pltpu.SemaphoreType.DMA for _ in range(2 * nbuf)]
        ),
        compiler_params=pltpu.CompilerParams(use_tc_tiling_on_sc=False),
    )
    def gather_kernel(idx_hbm, table_hbm, out_hbm, idx_v, *bufs_and_sems):
        rows = bufs_and_sems[:nbuf]
        gsem = bufs_and_sems[nbuf : 2 * nbuf]
        osem = bufs_and_sems[2 * nbuf :]
        wid = lax.axis_index("s") * _NUM_CORES + lax.axis_index("c")
        base = wid * rows_per_w
        pltpu.sync_copy(idx_hbm.at[pl.ds(base, rows_per_w)], idx_v)

        for b in range(nbuf):
            pltpu.async_copy(table_hbm.at[idx_v.at[b]], rows[b], gsem[b])

        def group(g, carry):
            for b in range(nbuf):
                j = g * nbuf + b
                pltpu.make_async_copy(
                    table_hbm.at[idx_v.at[j]], rows[b], gsem[b]
                ).wait()
                out_cp = pltpu.make_async_copy(
                    rows[b], out_hbm.at[base + j], osem[b]
                )
                out_cp.start()
                out_cp.wait()

                @pl.when(j + nbuf < rows_per_w)
                def _():
                    pltpu.async_copy(
                        table_hbm.at[idx_v.at[j + nbuf]], rows[b], gsem[b]
                    )

            return carry

        lax.fori_loop(0, n_groups, group, 0)

    return gather_kernel(input, table)


# speedup vs baseline: 1.0218x; 1.0218x over previous
import functools

import jax
import jax.numpy as jnp
from jax import lax
from jax.experimental import pallas as pl
from jax.experimental.pallas import tpu as pltpu
from jax.experimental.pallas import tpu_sc as plsc

_NUM_CORES = 2
_NUM_SUBCORES = 16
_NW = _NUM_CORES * _NUM_SUBCORES


_CHUNK = 256


def kernel(input, table):
    B, S = input.shape
    V, D = table.shape
    total = B * S
    per_w = total // _NW
    n_chunks = per_w // _CHUNK
    nbuf = 4
    n_groups = n_chunks // nbuf

    idx = input.T.reshape(_NW, n_chunks, _CHUNK)

    mesh = plsc.VectorSubcoreMesh(core_axis_name="c", subcore_axis_name="s")

    @functools.partial(
        pl.kernel,
        mesh=mesh,
        out_type=jax.ShapeDtypeStruct((_NW, n_chunks, _CHUNK, D), jnp.float32),
        scratch_types=(
            [pltpu.VMEM((n_chunks, _CHUNK), jnp.int32)]
            + [pltpu.VMEM((_CHUNK, D), jnp.float32) for _ in range(nbuf)]
            + [pltpu.SemaphoreType.DMA for _ in range(2 * nbuf)]
        ),
        compiler_params=pltpu.CompilerParams(use_tc_tiling_on_sc=False),
    )
    def gather_kernel(idx_hbm, table_hbm, out_hbm, idx_v, *bufs_and_sems):
        rows = bufs_and_sems[:nbuf]
        gsem = bufs_and_sems[nbuf : 2 * nbuf]
        osem = bufs_and_sems[2 * nbuf :]
        wid = lax.axis_index("s") * _NUM_CORES + lax.axis_index("c")
        pltpu.sync_copy(idx_hbm.at[wid], idx_v)

        for b in range(nbuf):
            pltpu.async_copy(table_hbm.at[idx_v.at[b]], rows[b], gsem[b])

        def group(g, carry):
            for b in range(nbuf):
                j = g * nbuf + b
                pltpu.make_async_copy(
                    table_hbm.at[idx_v.at[j]], rows[b], gsem[b]
                ).wait()
                out_cp = pltpu.make_async_copy(
                    rows[b], out_hbm.at[wid, j], osem[b]
                )
                out_cp.start()
                out_cp.wait()

                @pl.when(j + nbuf < n_chunks)
                def _():
                    pltpu.async_copy(
                        table_hbm.at[idx_v.at[j + nbuf]], rows[b], gsem[b]
                    )

            return carry

        lax.fori_loop(0, n_groups, group, 0)

    out = gather_kernel(idx, table)
    return out.reshape(S, B, D).transpose(1, 0, 2)
